# baseline (speedup 1.0000x reference)
import jax
import jax.numpy as jnp
from jax import lax
from jax.experimental import pallas as pl
from jax.experimental.pallas import tpu as pltpu
from jax.experimental.pallas import tpu_sc as plsc

N = 10000
E = 320000
RH = 128
R = 1000
RPAD = 1024
NC = 2
NS = 16
L = 16
EPT = E // NS
CB = 2000
KC = 80
NCK = EPT // KC


def _tc_proj_body(x_ref, wh_ref, wt_ref, a_ref, xr2_ref, st_ref):
    x = x_ref[...]
    wh = wh_ref[...]
    wt = wt_ref[...]
    xr2_ref[pl.ds(0, N), :] = jnp.dot(x, wh, preferred_element_type=jnp.float32)
    xr2_ref[pl.ds(N, N), :] = jnp.dot(x, wt, preferred_element_type=jnp.float32)
    a = a_ref[...]
    w0 = jnp.dot(wh, a[0], preferred_element_type=jnp.float32)
    w1 = jnp.dot(wt, a[1], preferred_element_type=jnp.float32)
    w2 = jnp.dot(wh, a[2], preferred_element_type=jnp.float32)
    w3 = jnp.dot(wt, a[3], preferred_element_type=jnp.float32)
    wc = jnp.stack([w0, w1, w2, w3], axis=0)
    st_ref[...] = lax.dot_general(
        wc, x, dimension_numbers=(((1,), (1,)), ((), ())),
        preferred_element_type=jnp.float32)


def _sc_body(xr2, st, eidx, rl, z64,
             out_hbm,
             s_a, s_b, myf, relf, oc, exb, psum, stot, iidx,
             relc0, relc1, relc2, rows0, rows1, rows2,
             gsem0, gsem1, gsem2, ssem0, ssem1, ssem2,
             sums_sh, out_sh):
    c = lax.axis_index("c")
    s = lax.axis_index("s")

    @pl.when(c == 0)
    def _():
        pltpu.sync_copy(st.at[0], s_a)
        pltpu.sync_copy(st.at[1], s_b)

    @pl.when(c == 1)
    def _():
        pltpu.sync_copy(st.at[3], s_a)
        pltpu.sync_copy(st.at[2], s_b)

    pltpu.sync_copy(eidx.at[pl.ds(c * E + s * EPT, EPT)], myf)
    pltpu.sync_copy(rl.at[pl.ds(s * EPT, EPT)], relf)

    pltpu.sync_copy(z64, out_sh.at[pl.ds(s * 64, 64)])
    pltpu.sync_copy(z64.at[pl.ds(0, 16)], psum)

    @pl.when(s == 0)
    def _():
        pltpu.sync_copy(z64.at[pl.ds(0, 16)], sums_sh)

    iidx[...] = lax.iota(jnp.int32, L)

    plsc.subcore_barrier()

    cN = c * N

    def _b_chunk(ci, _):
        base = s * EPT + ci * CB
        pltpu.sync_copy(eidx.at[pl.ds((1 - c) * E + base, CB)], oc)

        def _b_step(i, _):
            off = ci * CB + i * L
            mv = myf[pl.ds(off, L)]
            ov = oc[pl.ds(i * L, L)]
            rv = relf[pl.ds(off, L)]
            sa = plsc.load_gather(s_a, [mv])
            sb = plsc.load_gather(s_b, [ov])
            e = sa + sb
            lr = jnp.where(e > 0, e, e * jnp.float32(0.01))
            ex = jnp.exp(lr)
            exb[pl.ds(off, L)] = ex
            myf[pl.ds(off, L)] = mv + cN
            plsc.addupdate_scatter(psum, [rv >> 7, rv & 127], ex)
            return 0
        lax.fori_loop(0, CB // L, _b_step, 0)
        return 0
    lax.fori_loop(0, EPT // CB, _b_chunk, 0)

    pltpu.sync_copy(psum, sums_sh.at[iidx], add=True)
    plsc.subcore_barrier()

    pltpu.sync_copy(sums_sh, psum)

    def _red(rb, _):
        w16 = psum[rb >> 3, pl.ds((rb & 7) * L, L)]
        stot[pl.ds(rb * L, L)] = jnp.float32(1.0) / (w16 + jnp.float32(1e-16))
        return 0
    lax.fori_loop(0, RPAD // L, _red, 0)

    def _apre(i, _):
        off = i * L
        rv = relf[pl.ds(off, L)]
        sv = plsc.load_gather(stot, [rv])
        exb[pl.ds(off, L)] = exb[pl.ds(off, L)] * sv
        return 0
    lax.fori_loop(0, EPT // L, _apre, 0)

    relc = (relc0, relc1, relc2)
    rows = (rows0, rows1, rows2)
    gsem = (gsem0, gsem1, gsem2)
    ssem = (ssem0, ssem1, ssem2)

    def _prep(ci, b):
        off = ci * KC
        for k in range(KC // L):
            relc[b][pl.ds(k * L, L)] = relf[pl.ds(off + k * L, L)]
        idx = myf.at[pl.ds(off, KC)]
        pltpu.async_copy(xr2.at[idx.at[pl.ds(0, KC // 2)]],
                         rows[b].at[pl.ds(0, KC // 2)], gsem[b])
        pltpu.async_copy(xr2.at[idx.at[pl.ds(KC // 2, KC // 2)]],
                         rows[b].at[pl.ds(KC // 2, KC // 2)], gsem[b])

    def _gwait(b):
        pltpu.make_async_copy(xr2.at[pl.ds(0, KC)], rows[b], gsem[b]).wait()

    def _scale(ci, b):
        rows_b = rows[b]

        def body(ii, _):
            av = exb[pl.ds(ci * KC + ii * L, L)]
            for l in range(L):
                a = av[l]
                r = ii * L + l
                for j in range(RH // L):
                    rows_b[r, pl.ds(j * L, L)] = rows_b[r, pl.ds(j * L, L)] * a
            return 0
        lax.fori_loop(0, KC // L, body, 0)

    def _sstart(b):
        pltpu.async_copy(rows[b], out_sh.at[relc[b]], ssem[b], add=True)

    def _swait(b):
        pltpu.make_async_copy(rows[b], out_sh.at[pl.ds(0, KC)], ssem[b]).wait()

    _prep(0, 0)
    _prep(1, 1)
    _gwait(0)
    _prep(2, 2)
    _scale(0, 0)
    _sstart(0)

    def _c_triple(i3, _):
        for db in range(3):
            sl = (1 + db) % 3
            ci = 1 + 3 * i3 + db
            b2 = (sl + 2) % 3
            _gwait(sl)
            _swait(b2)
            _prep(ci + 2, b2)
            _scale(ci, sl)
            _sstart(sl)
        return 0
    lax.fori_loop(0, (NCK - 4) // 3, _c_triple, 0)

    _gwait(1)
    _swait(0)
    _prep(249, 0)
    _scale(247, 1)
    _sstart(1)

    _gwait(2)
    _swait(1)
    _scale(248, 2)
    _sstart(2)

    _gwait(0)
    _swait(2)
    _scale(249, 0)
    _sstart(0)
    _swait(0)

    plsc.subcore_barrier()

    off = c * RPAD + s * 64
    pltpu.sync_copy(out_sh.at[pl.ds(s * 64, 64)], out_hbm.at[pl.ds(off, 64)])


def _tc_add_body(o_ref, out_ref):
    out_ref[...] = o_ref[pl.ds(0, R), :] + o_ref[pl.ds(RPAD, R), :]


def kernel(x_e, edge_index, rel, rel_emb, r_index, line_graph_index,
           line_graph_val, W_h, W_t, a_h1, a_h2, a_t1, a_t2):
    a4 = jnp.stack([a_h1, a_h2, a_t1, a_t2], axis=0)

    xr2, st = pl.pallas_call(
        _tc_proj_body,
        out_shape=[
            jax.ShapeDtypeStruct((2 * N, RH), jnp.float32),
            jax.ShapeDtypeStruct((4, N), jnp.float32),
        ],
    )(x_e, W_h, W_t, a4)

    eidx = edge_index.reshape(2 * E)
    z64 = jnp.zeros((64, RH), jnp.float32)

    mesh = plsc.VectorSubcoreMesh(
        core_axis_name="c", subcore_axis_name="s", num_cores=NC,
        num_subcores=NS)
    sc = pl.kernel(
        _sc_body,
        out_type=jax.ShapeDtypeStruct((NC * RPAD, RH), jnp.float32),
        mesh=mesh,
        compiler_params=pltpu.CompilerParams(needs_layout_passes=False),
        scratch_types=[
            pltpu.VMEM((N,), jnp.float32),
            pltpu.VMEM((N,), jnp.float32),
            pltpu.VMEM((EPT,), jnp.int32),
            pltpu.VMEM((EPT,), jnp.int32),
            pltpu.VMEM((CB,), jnp.int32),
            pltpu.VMEM((EPT,), jnp.float32),
            pltpu.VMEM((16, 128), jnp.float32),
            pltpu.VMEM((RPAD,), jnp.float32),
            pltpu.VMEM((L,), jnp.int32),
            pltpu.VMEM((KC,), jnp.int32),
            pltpu.VMEM((KC,), jnp.int32),
            pltpu.VMEM((KC,), jnp.int32),
            pltpu.VMEM((KC, RH), jnp.float32),
            pltpu.VMEM((KC, RH), jnp.float32),
            pltpu.VMEM((KC, RH), jnp.float32),
            pltpu.SemaphoreType.DMA,
            pltpu.SemaphoreType.DMA,
            pltpu.SemaphoreType.DMA,
            pltpu.SemaphoreType.DMA,
            pltpu.SemaphoreType.DMA,
            pltpu.SemaphoreType.DMA,
            pltpu.VMEM_SHARED((16, 128), jnp.float32),
            pltpu.VMEM_SHARED((RPAD, RH), jnp.float32),
        ],
    )
    o2 = sc(xr2, st, eidx, rel, z64)

    out = pl.pallas_call(
        _tc_add_body,
        out_shape=jax.ShapeDtypeStruct((R, RH), jnp.float32),
    )(o2)
    return out


# speedup vs baseline: 1.0083x; 1.0083x over previous
import jax
import jax.numpy as jnp
from jax import lax
from jax.experimental import pallas as pl
from jax.experimental.pallas import tpu as pltpu
from jax.experimental.pallas import tpu_sc as plsc

N = 10000
E = 320000
RH = 128
R = 1000
RPAD = 1024
NC = 2
NS = 16
L = 16
EPT = E // NS
CB = 4000
KC = 80
NCK = EPT // KC


def _tc_proj_body(x_ref, wh_ref, wt_ref, a_ref, xr2_ref, st_ref):
    x = x_ref[...]
    wh = wh_ref[...]
    wt = wt_ref[...]
    xb = x.astype(jnp.bfloat16)
    xr2_ref[pl.ds(0, N), :] = jnp.dot(
        xb, wh.astype(jnp.bfloat16), preferred_element_type=jnp.float32)
    xr2_ref[pl.ds(N, N), :] = jnp.dot(
        xb, wt.astype(jnp.bfloat16), preferred_element_type=jnp.float32)
    a = a_ref[...]
    w0 = jnp.dot(wh, a[0], preferred_element_type=jnp.float32)
    w1 = jnp.dot(wt, a[1], preferred_element_type=jnp.float32)
    w2 = jnp.dot(wh, a[2], preferred_element_type=jnp.float32)
    w3 = jnp.dot(wt, a[3], preferred_element_type=jnp.float32)
    wc = jnp.stack([w0, w1, w2, w3], axis=0)
    st_ref[...] = lax.dot_general(
        wc, x, dimension_numbers=(((1,), (1,)), ((), ())),
        preferred_element_type=jnp.float32)


def _sc_body(xr2, st, eidx, rl, z64,
             out_hbm,
             s_a, s_b, myf, relf, oc, exb, psum, stot, iidx,
             relc0, relc1, relc2, rows0, rows1, rows2,
             gsem0, gsem1, gsem2, ssem0, ssem1, ssem2,
             sums_sh, out_sh):
    c = lax.axis_index("c")
    s = lax.axis_index("s")

    @pl.when(c == 0)
    def _():
        pltpu.sync_copy(st.at[0], s_a)
        pltpu.sync_copy(st.at[1], s_b)

    @pl.when(c == 1)
    def _():
        pltpu.sync_copy(st.at[3], s_a)
        pltpu.sync_copy(st.at[2], s_b)

    pltpu.sync_copy(eidx.at[pl.ds(c * E + s * EPT, EPT)], myf)
    pltpu.sync_copy(rl.at[pl.ds(s * EPT, EPT)], relf)

    pltpu.sync_copy(z64, out_sh.at[pl.ds(s * 64, 64)])
    pltpu.sync_copy(z64.at[pl.ds(0, 16)], psum)

    @pl.when(s == 0)
    def _():
        pltpu.sync_copy(z64.at[pl.ds(0, 16)], sums_sh)

    iidx[...] = lax.iota(jnp.int32, L)

    plsc.subcore_barrier()

    cN = c * N

    def _b_chunk(ci, _):
        base = s * EPT + ci * CB
        pltpu.sync_copy(eidx.at[pl.ds((1 - c) * E + base, CB)], oc)

        def _b_step(i, _):
            for u in range(2):
                off = ci * CB + i * (2 * L) + u * L
                mv = myf[pl.ds(off, L)]
                ov = oc[pl.ds(i * (2 * L) + u * L, L)]
                rv = relf[pl.ds(off, L)]
                sa = plsc.load_gather(s_a, [mv])
                sb = plsc.load_gather(s_b, [ov])
                e = sa + sb
                lr = jnp.where(e > 0, e, e * jnp.float32(0.01))
                ex = jnp.exp(lr)
                exb[pl.ds(off, L)] = ex
                myf[pl.ds(off, L)] = mv + cN
                plsc.addupdate_scatter(psum, [rv >> 7, rv & 127], ex)
            return 0
        lax.fori_loop(0, CB // (2 * L), _b_step, 0)
        return 0
    lax.fori_loop(0, EPT // CB, _b_chunk, 0)

    pltpu.sync_copy(psum, sums_sh.at[iidx], add=True)
    plsc.subcore_barrier()

    pltpu.sync_copy(sums_sh, psum)

    def _red(rb, _):
        w16 = psum[rb >> 3, pl.ds((rb & 7) * L, L)]
        stot[pl.ds(rb * L, L)] = jnp.float32(1.0) / (w16 + jnp.float32(1e-16))
        return 0
    lax.fori_loop(0, RPAD // L, _red, 0)

    def _apre(i, _):
        for u in range(2):
            off = i * (2 * L) + u * L
            rv = relf[pl.ds(off, L)]
            sv = plsc.load_gather(stot, [rv])
            exb[pl.ds(off, L)] = exb[pl.ds(off, L)] * sv
        return 0
    lax.fori_loop(0, EPT // (2 * L), _apre, 0)

    relc = (relc0, relc1, relc2)
    rows = (rows0, rows1, rows2)
    gsem = (gsem0, gsem1, gsem2)
    ssem = (ssem0, ssem1, ssem2)

    def _prep(ci, b):
        off = ci * KC
        for k in range(KC // L):
            relc[b][pl.ds(k * L, L)] = relf[pl.ds(off + k * L, L)]
        pltpu.async_copy(xr2.at[myf.at[pl.ds(off, KC)]], rows[b], gsem[b])

    def _gwait(b):
        pltpu.make_async_copy(xr2.at[pl.ds(0, KC)], rows[b], gsem[b]).wait()

    def _scale(ci, b):
        rows_b = rows[b]

        def body(ii, _):
            av = exb[pl.ds(ci * KC + ii * L, L)]
            for l in range(L):
                a = av[l]
                r = ii * L + l
                for j in range(RH // L):
                    rows_b[r, pl.ds(j * L, L)] = rows_b[r, pl.ds(j * L, L)] * a
            return 0
        lax.fori_loop(0, KC // L, body, 0)

    def _sstart(b):
        pltpu.async_copy(rows[b], out_sh.at[relc[b]], ssem[b], add=True)

    def _swait(b):
        pltpu.make_async_copy(rows[b], out_sh.at[pl.ds(0, KC)], ssem[b]).wait()

    _prep(0, 0)
    _prep(1, 1)
    _gwait(0)
    _prep(2, 2)
    _scale(0, 0)
    _sstart(0)

    def _c_triple(i3, _):
        for db in range(3):
            sl = (1 + db) % 3
            ci = 1 + 3 * i3 + db
            b2 = (sl + 2) % 3
            _gwait(sl)
            _swait(b2)
            _prep(ci + 2, b2)
            _scale(ci, sl)
            _sstart(sl)
        return 0
    lax.fori_loop(0, (NCK - 4) // 3, _c_triple, 0)

    _gwait(1)
    _swait(0)
    _prep(249, 0)
    _scale(247, 1)
    _sstart(1)

    _gwait(2)
    _swait(1)
    _scale(248, 2)
    _sstart(2)

    _gwait(0)
    _swait(2)
    _scale(249, 0)
    _sstart(0)
    _swait(0)

    plsc.subcore_barrier()

    off = c * RPAD + s * 64
    pltpu.sync_copy(out_sh.at[pl.ds(s * 64, 64)], out_hbm.at[pl.ds(off, 64)])


def _tc_add_body(o_ref, out_ref):
    out_ref[...] = o_ref[pl.ds(0, R), :] + o_ref[pl.ds(RPAD, R), :]


def kernel(x_e, edge_index, rel, rel_emb, r_index, line_graph_index,
           line_graph_val, W_h, W_t, a_h1, a_h2, a_t1, a_t2):
    a4 = jnp.stack([a_h1, a_h2, a_t1, a_t2], axis=0)

    xr2, st = pl.pallas_call(
        _tc_proj_body,
        out_shape=[
            jax.ShapeDtypeStruct((2 * N, RH), jnp.float32),
            jax.ShapeDtypeStruct((4, N), jnp.float32),
        ],
    )(x_e, W_h, W_t, a4)

    eidx = edge_index.reshape(2 * E)
    z64 = jnp.zeros((64, RH), jnp.float32)

    mesh = plsc.VectorSubcoreMesh(
        core_axis_name="c", subcore_axis_name="s", num_cores=NC,
        num_subcores=NS)
    sc = pl.kernel(
        _sc_body,
        out_type=jax.ShapeDtypeStruct((NC * RPAD, RH), jnp.float32),
        mesh=mesh,
        compiler_params=pltpu.CompilerParams(needs_layout_passes=False),
        scratch_types=[
            pltpu.VMEM((N,), jnp.float32),
            pltpu.VMEM((N,), jnp.float32),
            pltpu.VMEM((EPT,), jnp.int32),
            pltpu.VMEM((EPT,), jnp.int32),
            pltpu.VMEM((CB,), jnp.int32),
            pltpu.VMEM((EPT,), jnp.float32),
            pltpu.VMEM((16, 128), jnp.float32),
            pltpu.VMEM((RPAD,), jnp.float32),
            pltpu.VMEM((L,), jnp.int32),
            pltpu.VMEM((KC,), jnp.int32),
            pltpu.VMEM((KC,), jnp.int32),
            pltpu.VMEM((KC,), jnp.int32),
            pltpu.VMEM((KC, RH), jnp.float32),
            pltpu.VMEM((KC, RH), jnp.float32),
            pltpu.VMEM((KC, RH), jnp.float32),
            pltpu.SemaphoreType.DMA,
            pltpu.SemaphoreType.DMA,
            pltpu.SemaphoreType.DMA,
            pltpu.SemaphoreType.DMA,
            pltpu.SemaphoreType.DMA,
            pltpu.SemaphoreType.DMA,
            pltpu.VMEM_SHARED((16, 128), jnp.float32),
            pltpu.VMEM_SHARED((RPAD, RH), jnp.float32),
        ],
    )
    o2 = sc(xr2, st, eidx, rel, z64)

    out = pl.pallas_call(
        _tc_add_body,
        out_shape=jax.ShapeDtypeStruct((R, RH), jnp.float32),
    )(o2)
    return out
